# baseline (speedup 1.0000x reference)
import functools

import jax
import jax.numpy as jnp
from jax import lax
from jax.experimental import pallas as pl
from jax.experimental.pallas import tpu as pltpu
from jax.experimental.pallas import tpu_sc as plsc

TOKENS = 32768
HIDDEN = 4096
N_EXPERTS = 64
TOP_K = 8
TILE = 1024

NUM_WORKERS = 32
LANES = 16
TOK_PER_W = TOKENS // NUM_WORKERS
GROUPS = TOK_PER_W // LANES


def _probs_kernel(x_ref, w_ref, probs_ref):
    logits = jax.lax.dot_general(
        x_ref[...],
        w_ref[...],
        (((1,), (1,)), ((), ())),
        preferred_element_type=jnp.float32,
    )
    m = jnp.max(logits, axis=-1, keepdims=True)
    e = jnp.exp(logits - m)
    s = jnp.sum(e, axis=-1, keepdims=True)
    probs_ref[...] = e / s


_sc_mesh = plsc.VectorSubcoreMesh(core_axis_name="c", subcore_axis_name="s")


@functools.partial(
    pl.kernel,
    out_type=[
        jax.ShapeDtypeStruct((TOKENS * TOP_K,), jnp.float32),
        jax.ShapeDtypeStruct((TOKENS * TOP_K,), jnp.int32),
    ],
    mesh=_sc_mesh,
    compiler_params=pltpu.CompilerParams(needs_layout_passes=False),
    scratch_types=[
        pltpu.VMEM((TOK_PER_W * N_EXPERTS,), jnp.float32),
        pltpu.VMEM((TOK_PER_W * TOP_K,), jnp.float32),
        pltpu.VMEM((TOK_PER_W * TOP_K,), jnp.int32),
    ],
)
def _sc_topk(probs_hbm, vals_hbm, idx_hbm, p_v, v_v, i_v):
    wid = lax.axis_index("s") * 2 + lax.axis_index("c")
    base = wid * TOK_PER_W
    pltpu.sync_copy(
        probs_hbm.at[pl.ds(base * N_EXPERTS, TOK_PER_W * N_EXPERTS)], p_v
    )

    lane = lax.iota(jnp.int32, LANES)

    def body(g, carry):
        rows = jnp.full((LANES,), g * LANES, jnp.int32) + lane
        neg = jnp.full((LANES,), -jnp.inf, jnp.float32)
        zero_i = jnp.zeros((LANES,), jnp.int32)
        tv = [neg] * TOP_K
        ti = [zero_i] * TOP_K
        for e in range(N_EXPERTS):
            cols = jnp.full((LANES,), e, jnp.int32)
            v = plsc.load_gather(p_v, [rows * N_EXPERTS + cols])
            ie = cols
            for j in range(TOP_K):
                c = v > tv[j]
                tv_new = jnp.where(c, v, tv[j])
                ti_new = jnp.where(c, ie, ti[j])
                v = jnp.where(c, tv[j], v)
                ie = jnp.where(c, ti[j], ie)
                tv[j] = tv_new
                ti[j] = ti_new
        s = tv[0]
        for j in range(1, TOP_K):
            s = s + tv[j]
        inv = jnp.full((LANES,), 1.0, jnp.float32) / s
        out_base = rows * TOP_K
        for j in range(TOP_K):
            jcol = jnp.full((LANES,), j, jnp.int32)
            plsc.store_scatter(v_v, [out_base + jcol], tv[j] * inv)
            plsc.store_scatter(i_v, [out_base + jcol], ti[j])
        return carry

    lax.fori_loop(0, GROUPS, body, 0)

    pltpu.sync_copy(v_v, vals_hbm.at[pl.ds(base * TOP_K, TOK_PER_W * TOP_K)])
    pltpu.sync_copy(i_v, idx_hbm.at[pl.ds(base * TOP_K, TOK_PER_W * TOP_K)])


@jax.jit
def kernel(x, W):
    n_tiles = TOKENS // TILE
    probs = pl.pallas_call(
        _probs_kernel,
        grid=(n_tiles,),
        in_specs=[
            pl.BlockSpec((TILE, HIDDEN), lambda i: (i, 0)),
            pl.BlockSpec((N_EXPERTS, HIDDEN), lambda i: (0, 0)),
        ],
        out_specs=pl.BlockSpec((TILE, N_EXPERTS), lambda i: (i, 0)),
        out_shape=jax.ShapeDtypeStruct((TOKENS, N_EXPERTS), jnp.float32),
        compiler_params=pltpu.CompilerParams(
            dimension_semantics=("parallel",),
        ),
    )(x, W)
    top_vals, top_idx = _sc_topk(probs.reshape(-1))
    return (
        probs,
        top_vals.reshape(TOKENS, TOP_K),
        top_idx.reshape(TOKENS, TOP_K),
    )


# speedup vs baseline: 1.4077x; 1.4077x over previous
import jax
import jax.numpy as jnp
from jax.experimental import pallas as pl
from jax.experimental.pallas import tpu as pltpu

TOKENS = 32768
HIDDEN = 4096
N_EXPERTS = 64
TOP_K = 8
TILE = 1024


def _gate_kernel(x_ref, w_ref, probs_ref, vals_ref, idx_ref):
    x = x_ref[...]
    w = w_ref[...]
    logits = jax.lax.dot_general(
        x, w, (((1,), (1,)), ((), ())), preferred_element_type=jnp.float32
    )
    m = jnp.max(logits, axis=-1, keepdims=True)
    e = jnp.exp(logits - m)
    s = jnp.sum(e, axis=-1, keepdims=True)
    probs = e / s
    probs_ref[...] = probs

    work = probs
    iota = jax.lax.broadcasted_iota(jnp.int32, probs.shape, 1).astype(
        jnp.float32
    )
    vals = []
    idxs = []
    for _ in range(TOP_K):
        v = jnp.max(work, axis=-1, keepdims=True)
        i = jnp.min(
            jnp.where(work == v, iota, float(N_EXPERTS)),
            axis=-1,
            keepdims=True,
        )
        vals.append(v)
        idxs.append(i)
        work = jnp.where(iota == i, -jnp.inf, work)
    top_vals = jnp.concatenate(vals, axis=-1)
    top_idx = jnp.concatenate(idxs, axis=-1)
    top_vals = top_vals / jnp.sum(top_vals, axis=-1, keepdims=True)
    vals_ref[...] = top_vals
    idx_ref[...] = top_idx.astype(jnp.int32)


@jax.jit
def kernel(x, W):
    n_tiles = TOKENS // TILE
    probs, top_vals, top_idx = pl.pallas_call(
        _gate_kernel,
        grid=(n_tiles,),
        in_specs=[
            pl.BlockSpec((TILE, HIDDEN), lambda i: (i, 0)),
            pl.BlockSpec((N_EXPERTS, HIDDEN), lambda i: (0, 0)),
        ],
        out_specs=[
            pl.BlockSpec((TILE, N_EXPERTS), lambda i: (i, 0)),
            pl.BlockSpec((TILE, TOP_K), lambda i: (i, 0)),
            pl.BlockSpec((TILE, TOP_K), lambda i: (i, 0)),
        ],
        out_shape=[
            jax.ShapeDtypeStruct((TOKENS, N_EXPERTS), jnp.float32),
            jax.ShapeDtypeStruct((TOKENS, TOP_K), jnp.float32),
            jax.ShapeDtypeStruct((TOKENS, TOP_K), jnp.int32),
        ],
        compiler_params=pltpu.CompilerParams(
            dimension_semantics=("parallel",),
        ),
    )(x, W)
    return (probs, top_vals, top_idx)
